# baseline (speedup 1.0000x reference)
import functools

import jax
import jax.numpy as jnp
from jax import lax
from jax.experimental import pallas as pl
from jax.experimental.pallas import tpu as pltpu
from jax.experimental.pallas import tpu_sc as plsc

_N = 4096
_B = 4
_R = 256
_C = 1024
_BIG = 1e10
_L = 16
_NP = _N + 128


def _sc_compact_body(predT_hbm, targetT_hbm, pointsT_hbm, mask_hbm,
                     aCT_hbm, bCT_hbm, cnt_hbm, l1_hbm,
                     src1, src2, maskv, posbuf, valbuf, zbuf,
                     stage_i, stage_f, shared, sem):
    wid = lax.axis_index("s") * 2 + lax.axis_index("c")
    batch = wid // 8
    role = wid % 8
    n_chunks = _N // _L

    liota = lax.iota(jnp.int32, _L)
    _dnums = lax.GatherDimensionNumbers(
        offset_dims=(), collapsed_slice_dims=(0,), start_index_map=(0,))

    def _vgather(x, idx):
        return lax.gather(x, idx[:, None], _dnums, slice_sizes=(1,),
                          mode=lax.GatherScatterMode.PROMISE_IN_BOUNDS)

    def _prefix_sum(x):
        for k in (1, 2, 4, 8):
            idx = jnp.maximum(liota - k, 0)
            g = _vgather(x, idx)
            x = x + jnp.where(liota >= k, g, 0)
        return x

    def _zero_fill():
        def zb(i, c):
            zbuf[pl.ds(i * _L, _L)] = jnp.zeros((_L,), jnp.float32)
            return c
        lax.fori_loop(0, _NP // _L, zb, 0, unroll=8)

    @pl.when(role < 7)
    def _load_mask():
        pltpu.sync_copy(mask_hbm.at[batch, :], maskv)

    def _compact(src2_hbm, dst_hbm, coord, emit_cnt):
        pltpu.sync_copy(pointsT_hbm.at[batch * 3 + coord, :], src1)
        pltpu.sync_copy(src2_hbm.at[batch * 3 + coord, :], src2)
        row0 = (batch * 8 + coord) * _NP
        sh0 = lax.axis_index("s") * _NP
        _zero_fill()
        pltpu.sync_copy(zbuf, shared.at[pl.ds(sh0, _NP)])

        def cbody(i, base):
            c0 = i * _L
            mv = maskv[pl.ds(c0, _L)]
            valid = mv > 0
            x = src1[pl.ds(c0, _L)] + src2[pl.ds(c0, _L)]
            pre = _prefix_sum(mv)
            pos = jnp.where(valid, base + pre - 1, _N + liota) + sh0
            posbuf[0, pl.ds(c0, _L)] = pos
            valbuf[0, pl.ds(c0, _L)] = x
            total = _vgather(pre, jnp.full((_L,), _L - 1, jnp.int32))
            return base + total
        cnt = lax.fori_loop(0, n_chunks, cbody, jnp.zeros((_L,), jnp.int32),
                            unroll=4)

        copies = []
        for j in range(_N // 128):
            pb = posbuf.at[0, pl.ds(j * 128, 128)]
            vb = valbuf.at[0, pl.ds(j * 128, 128)]
            copies.append(pltpu.async_copy(vb, shared.at[pb], sem))
        for c in copies:
            c.wait()
        pltpu.sync_copy(shared.at[pl.ds(sh0, _N)], src1)
        pltpu.sync_copy(src1, dst_hbm.at[pl.ds(row0, _N)])

        if emit_cnt:
            stage_i[pl.ds(0, _L)] = cnt
            pltpu.sync_copy(stage_i, cnt_hbm.at[batch, :])

    @pl.when(role < 3)
    def _a_side():
        _compact(targetT_hbm, aCT_hbm, role, False)

    @pl.when(role == 3)
    def _b_side0():
        _compact(predT_hbm, bCT_hbm, jnp.int32(0), True)

    @pl.when(jnp.logical_and(role >= 4, role < 6))
    def _b_side():
        _compact(predT_hbm, bCT_hbm, role - 3, False)

    @pl.when(role == 6)
    def _l1_and_zero_a():
        acc0 = jnp.zeros((_L,), jnp.float32)
        for k in range(3):
            pltpu.sync_copy(predT_hbm.at[batch * 3 + k, :], src1)
            pltpu.sync_copy(targetT_hbm.at[batch * 3 + k, :], src2)

            def l1body(i, acc):
                c0 = i * _L
                mv = maskv[pl.ds(c0, _L)].astype(jnp.float32)
                diff = jnp.abs(src1[pl.ds(c0, _L)] - src2[pl.ds(c0, _L)])
                return acc + diff * mv
            acc0 = lax.fori_loop(0, n_chunks, l1body, acc0, unroll=4)
        stage_f[pl.ds(0, _L)] = acc0
        pltpu.sync_copy(stage_f, l1_hbm.at[batch, :])

        _zero_fill()
        for k in range(3, 8):
            pltpu.sync_copy(zbuf, aCT_hbm.at[pl.ds((batch * 8 + k) * _NP,
                                                   _NP)])

    @pl.when(role == 7)
    def _zero_b():
        _zero_fill()
        for k in range(3, 8):
            pltpu.sync_copy(zbuf, bCT_hbm.at[pl.ds((batch * 8 + k) * _NP,
                                                   _NP)])


def _sc_compact(predT, targetT, pointsT, mask):
    return functools.partial(
        pl.kernel,
        mesh=plsc.VectorSubcoreMesh(core_axis_name="c", subcore_axis_name="s",
                                    num_cores=2, num_subcores=16),
        out_type=[
            jax.ShapeDtypeStruct((_B * 8 * _NP,), jnp.float32),
            jax.ShapeDtypeStruct((_B * 8 * _NP,), jnp.float32),
            jax.ShapeDtypeStruct((_B, _L), jnp.int32),
            jax.ShapeDtypeStruct((_B, _L), jnp.float32),
        ],
        scratch_types=[
            pltpu.VMEM((_N,), jnp.float32),
            pltpu.VMEM((_N,), jnp.float32),
            pltpu.VMEM((_N,), jnp.int32),
            pltpu.VMEM((1, _N), jnp.int32),
            pltpu.VMEM((1, _N), jnp.float32),
            pltpu.VMEM((_NP,), jnp.float32),
            pltpu.VMEM((_L,), jnp.int32),
            pltpu.VMEM((_L,), jnp.float32),
            pltpu.VMEM_SHARED((16 * _NP,), jnp.float32),
            pltpu.SemaphoreType.DMA,
        ],
    )(_sc_compact_body)(predT, targetT, pointsT, mask)


def _tc_chamfer_kernel(aCT_ref, bCT_ref, cnts_ref, l1s_ref, out_ref,
                       aneg_s, bneg_s, ca_s, cb_s, aN_s, bN_s, acc_ref,
                       *, n_batch):
    b = pl.program_id(0)
    cnt_i = cnts_ref[b, 0]
    cntf = cnt_i.astype(jnp.float32)

    @pl.when(b == 0)
    def _init():
        acc_ref[0] = 0.0
        acc_ref[1] = 0.0
        acc_ref[2] = 0.0

    apt = aCT_ref[0][:, :_N]
    bpt = bCT_ref[0][:, :_N]
    a2r = jnp.sum(apt * apt, axis=0, keepdims=True)
    b2r = jnp.sum(bpt * bpt, axis=0, keepdims=True)
    aneg_s[:, :] = -2.0 * apt
    bneg_s[:, :] = -2.0 * bpt
    iota = lax.broadcasted_iota(jnp.int32, (1, _N), 1)
    pad = jnp.where(iota < cnt_i, 0.0, _BIG)
    ca_s[:, :] = a2r + pad
    cb_s[:, :] = b2r + pad
    aN_s[:, :] = jnp.transpose(apt, (1, 0))
    bN_s[:, :] = jnp.transpose(bpt, (1, 0))

    nr = (cnt_i + 2 * _R - 1) // (2 * _R)
    nc = (cnt_i + 2 * _C - 1) // (2 * _C)

    def _direction(rowN_s, colneg_s, cmask_s):
        def row_body(i, acc):
            blks = []
            for v in range(2):
                r0 = i * (2 * _R) + v * _R
                blk = rowN_s[pl.ds(r0, _R), :]
                r2 = jnp.sum(blk * blk, axis=1, keepdims=True)
                riota = lax.broadcasted_iota(jnp.int32, (_R, 1), 0) + r0
                rvalid = riota < cnt_i
                blks.append((blk, r2, rvalid))

            def col_body(j, rvmins):
                rv0, rv1 = rvmins
                c0 = j * (2 * _C)
                for u in range(2):
                    cu = c0 + u * _C
                    cn = colneg_s[:, pl.ds(cu, _C)]
                    cm = cmask_s[:, pl.ds(cu, _C)]
                    for v, (blk, r2, rvalid) in enumerate(blks):
                        s = lax.dot_general(
                            blk, cn, (((1,), (0,)), ((), ())),
                            preferred_element_type=jnp.float32)
                        rv = jnp.min(s + cm, axis=1, keepdims=True)
                        if v == 0:
                            rv0 = jnp.minimum(rv0, rv)
                        else:
                            rv1 = jnp.minimum(rv1, rv)
                return rv0, rv1

            init = jnp.full((_R, 1), _BIG, jnp.float32)
            rv0, rv1 = lax.fori_loop(0, nc, col_body, (init, init))
            part = jnp.float32(0.0)
            for (blk, r2, rvalid), rv in zip(blks, (rv0, rv1)):
                mrow = jnp.where(rvalid, 1.0, 0.0)
                part = part + jnp.sum(jnp.maximum(rv + r2, 0.0) * mrow)
            return acc + part

        return lax.fori_loop(0, nr, row_body, jnp.float32(0.0))

    sum_ab = _direction(aN_s, bneg_s, cb_s)
    sum_ba = _direction(bN_s, aneg_s, ca_s)
    cd_b = (sum_ab + sum_ba) / cntf

    l1_b = l1s_ref[b, 0]
    for k in range(1, _L):
        l1_b = l1_b + l1s_ref[b, k]
    acc_ref[0] = acc_ref[0] + l1_b
    acc_ref[1] = acc_ref[1] + cntf
    acc_ref[2] = acc_ref[2] + cd_b

    @pl.when(b == n_batch - 1)
    def _emit():
        l1 = acc_ref[0] / (3.0 * acc_ref[1])
        cd = acc_ref[2] * (1.0 / n_batch)
        out_ref[0, 0] = 0.5 * (l1 + cd)


def _tc_chamfer(aCT, bCT, cnts, l1s, n_batch):
    return pl.pallas_call(
        functools.partial(_tc_chamfer_kernel, n_batch=n_batch),
        grid=(n_batch,),
        in_specs=[
            pl.BlockSpec((1, 8, _NP), lambda b: (b, 0, 0)),
            pl.BlockSpec((1, 8, _NP), lambda b: (b, 0, 0)),
            pl.BlockSpec(memory_space=pltpu.SMEM),
            pl.BlockSpec(memory_space=pltpu.SMEM),
        ],
        out_specs=pl.BlockSpec((1, 1), lambda b: (0, 0),
                               memory_space=pltpu.SMEM),
        out_shape=jax.ShapeDtypeStruct((1, 1), jnp.float32),
        scratch_shapes=[
            pltpu.VMEM((8, _N), jnp.float32),
            pltpu.VMEM((8, _N), jnp.float32),
            pltpu.VMEM((1, _N), jnp.float32),
            pltpu.VMEM((1, _N), jnp.float32),
            pltpu.VMEM((_N, 8), jnp.float32),
            pltpu.VMEM((_N, 8), jnp.float32),
            pltpu.SMEM((4,), jnp.float32),
        ],
    )(aCT, bCT, cnts, l1s)


@jax.jit
def kernel(pred, target, mask, points):
    B, N, D = pred.shape
    predT = jnp.transpose(pred, (0, 2, 1)).reshape(B * D, N)
    targetT = jnp.transpose(target, (0, 2, 1)).reshape(B * D, N)
    pointsT = jnp.transpose(points, (0, 2, 1)).reshape(B * D, N)
    aCT_f, bCT_f, cnts, l1s = _sc_compact(predT, targetT, pointsT, mask)
    aCT = aCT_f.reshape(B, 8, _NP)
    bCT = bCT_f.reshape(B, 8, _NP)
    out = _tc_chamfer(aCT, bCT, cnts, l1s, B)
    return out[0, 0]


# speedup vs baseline: 1.3506x; 1.3506x over previous
import functools

import jax
import jax.numpy as jnp
from jax import lax
from jax.experimental import pallas as pl
from jax.experimental.pallas import tpu as pltpu
from jax.experimental.pallas import tpu_sc as plsc

_N = 4096
_B = 4
_R = 256
_C = 1024
_BIG = 1e10
_L = 16
_NP = _N + 128


def _sc_compact_body(predT_hbm, targetT_hbm, pointsT_hbm, mask_hbm,
                     aCT_hbm, bCT_hbm, cnt_hbm, l1_hbm,
                     src1, src2, maskv, posbuf, valbuf, zbuf,
                     stage_i, stage_f, shared, sem):
    wid = lax.axis_index("s") * 2 + lax.axis_index("c")
    batch = wid // 8
    role = wid % 8
    n_chunks = _N // _L

    liota = lax.iota(jnp.int32, _L)
    _dnums = lax.GatherDimensionNumbers(
        offset_dims=(), collapsed_slice_dims=(0,), start_index_map=(0,))

    def _vgather(x, idx):
        return lax.gather(x, idx[:, None], _dnums, slice_sizes=(1,),
                          mode=lax.GatherScatterMode.PROMISE_IN_BOUNDS)

    def _prefix_sum(x):
        for k in (1, 2, 4, 8):
            idx = jnp.maximum(liota - k, 0)
            g = _vgather(x, idx)
            x = x + jnp.where(liota >= k, g, 0)
        return x

    def _zero_fill():
        def zb(i, c):
            zbuf[pl.ds(i * _L, _L)] = jnp.zeros((_L,), jnp.float32)
            return c
        lax.fori_loop(0, _NP // _L, zb, 0, unroll=8)

    @pl.when(role < 7)
    def _load_mask():
        pltpu.sync_copy(mask_hbm.at[batch, :], maskv)

    def _compact(src2_hbm, dst_hbm, coord, emit_cnt):
        pltpu.sync_copy(pointsT_hbm.at[batch * 3 + coord, :], src1)
        pltpu.sync_copy(src2_hbm.at[batch * 3 + coord, :], src2)
        row0 = (batch * 8 + coord) * _NP
        sh0 = lax.axis_index("s") * _NP
        _zero_fill()
        pltpu.sync_copy(zbuf, shared.at[pl.ds(sh0, _NP)])

        def cbody(i, base):
            c0 = i * _L
            mv = maskv[pl.ds(c0, _L)]
            valid = mv > 0
            x = src1[pl.ds(c0, _L)] + src2[pl.ds(c0, _L)]
            pre = _prefix_sum(mv)
            pos = jnp.where(valid, base + pre - 1, _N + liota) + sh0
            posbuf[0, pl.ds(c0, _L)] = pos
            valbuf[0, pl.ds(c0, _L)] = x
            total = _vgather(pre, jnp.full((_L,), _L - 1, jnp.int32))
            return base + total
        cnt = lax.fori_loop(0, n_chunks, cbody, jnp.zeros((_L,), jnp.int32),
                            unroll=4)

        copies = []
        for j in range(_N // 128):
            pb = posbuf.at[0, pl.ds(j * 128, 128)]
            vb = valbuf.at[0, pl.ds(j * 128, 128)]
            copies.append(pltpu.async_copy(vb, shared.at[pb], sem))
        for c in copies:
            c.wait()
        pltpu.sync_copy(shared.at[pl.ds(sh0, _N)], src1)
        pltpu.sync_copy(src1, dst_hbm.at[pl.ds(row0, _N)])

        if emit_cnt:
            stage_i[pl.ds(0, _L)] = cnt
            pltpu.sync_copy(stage_i, cnt_hbm.at[batch, :])

    @pl.when(role < 3)
    def _a_side():
        _compact(targetT_hbm, aCT_hbm, role, False)

    @pl.when(role == 3)
    def _b_side0():
        _compact(predT_hbm, bCT_hbm, jnp.int32(0), True)

    @pl.when(jnp.logical_and(role >= 4, role < 6))
    def _b_side():
        _compact(predT_hbm, bCT_hbm, role - 3, False)

    @pl.when(role == 6)
    def _l1_and_zero_a():
        acc0 = jnp.zeros((_L,), jnp.float32)
        for k in range(3):
            pltpu.sync_copy(predT_hbm.at[batch * 3 + k, :], src1)
            pltpu.sync_copy(targetT_hbm.at[batch * 3 + k, :], src2)

            def l1body(i, acc):
                c0 = i * _L
                mv = maskv[pl.ds(c0, _L)].astype(jnp.float32)
                diff = jnp.abs(src1[pl.ds(c0, _L)] - src2[pl.ds(c0, _L)])
                return acc + diff * mv
            acc0 = lax.fori_loop(0, n_chunks, l1body, acc0, unroll=4)
        stage_f[pl.ds(0, _L)] = acc0
        pltpu.sync_copy(stage_f, l1_hbm.at[batch, :])

        _zero_fill()
        for k in range(3, 8):
            pltpu.sync_copy(zbuf, aCT_hbm.at[pl.ds((batch * 8 + k) * _NP,
                                                   _NP)])

    @pl.when(role == 7)
    def _zero_b():
        _zero_fill()
        for k in range(3, 8):
            pltpu.sync_copy(zbuf, bCT_hbm.at[pl.ds((batch * 8 + k) * _NP,
                                                   _NP)])


def _sc_compact(predT, targetT, pointsT, mask):
    return functools.partial(
        pl.kernel,
        mesh=plsc.VectorSubcoreMesh(core_axis_name="c", subcore_axis_name="s",
                                    num_cores=2, num_subcores=16),
        out_type=[
            jax.ShapeDtypeStruct((_B * 8 * _NP,), jnp.float32),
            jax.ShapeDtypeStruct((_B * 8 * _NP,), jnp.float32),
            jax.ShapeDtypeStruct((_B, _L), jnp.int32),
            jax.ShapeDtypeStruct((_B, _L), jnp.float32),
        ],
        scratch_types=[
            pltpu.VMEM((_N,), jnp.float32),
            pltpu.VMEM((_N,), jnp.float32),
            pltpu.VMEM((_N,), jnp.int32),
            pltpu.VMEM((1, _N), jnp.int32),
            pltpu.VMEM((1, _N), jnp.float32),
            pltpu.VMEM((_NP,), jnp.float32),
            pltpu.VMEM((_L,), jnp.int32),
            pltpu.VMEM((_L,), jnp.float32),
            pltpu.VMEM_SHARED((16 * _NP,), jnp.float32),
            pltpu.SemaphoreType.DMA,
        ],
    )(_sc_compact_body)(predT, targetT, pointsT, mask)


def _tc_chamfer_kernel(aCT_ref, bCT_ref, cnts_ref, l1s_ref, out_ref,
                       bneg_s, cb_s, colmin_s, aN_s, acc_ref, *, n_batch):
    b = pl.program_id(0)
    cnt_i = cnts_ref[b, 0]
    cntf = cnt_i.astype(jnp.float32)

    @pl.when(b == 0)
    def _init():
        acc_ref[0] = 0.0
        acc_ref[1] = 0.0
        acc_ref[2] = 0.0

    bpt = bCT_ref[0][:, :_N]
    b2 = jnp.sum(bpt * bpt, axis=0, keepdims=True)
    bneg_s[:, :] = -2.0 * bpt
    iota = lax.broadcasted_iota(jnp.int32, (1, _N), 1)
    cb_s[:, :] = b2 + jnp.where(iota < cnt_i, 0.0, _BIG)
    colmin_s[:, :] = jnp.full((1, _N), _BIG, jnp.float32)
    aN_s[:, :] = jnp.transpose(aCT_ref[0][:, :_N], (1, 0))

    nr = (cnt_i + 2 * _R - 1) // (2 * _R)
    nc = (cnt_i + 2 * _C - 1) // (2 * _C)

    def row_body(i, sum_ab):
        blks = []
        for v in range(2):
            r0 = i * (2 * _R) + v * _R
            a_blk = aN_s[pl.ds(r0, _R), :]
            a2 = jnp.sum(a_blk * a_blk, axis=1, keepdims=True)
            riota = lax.broadcasted_iota(jnp.int32, (_R, 1), 0) + r0
            rvalid = riota < cnt_i
            ca = a2 + jnp.where(rvalid, 0.0, _BIG)
            blks.append((a_blk, a2, rvalid, ca))

        def col_body(j, rvmins):
            rv0, rv1 = rvmins
            c0 = j * (2 * _C)
            for u in range(2):
                cu = c0 + u * _C
                bn = bneg_s[:, pl.ds(cu, _C)]
                cbj = cb_s[:, pl.ds(cu, _C)]
                cvs = []
                for v, (a_blk, a2, rvalid, ca) in enumerate(blks):
                    s = lax.dot_general(a_blk, bn, (((1,), (0,)), ((), ())),
                                        preferred_element_type=jnp.float32)
                    rv = jnp.min(s + cbj, axis=1, keepdims=True)
                    if v == 0:
                        rv0 = jnp.minimum(rv0, rv)
                    else:
                        rv1 = jnp.minimum(rv1, rv)
                    cvs.append(jnp.min(s + ca, axis=0, keepdims=True))
                cv = jnp.minimum(cvs[0], cvs[1])
                colmin_s[:, pl.ds(cu, _C)] = jnp.minimum(
                    colmin_s[:, pl.ds(cu, _C)], cv)
            return rv0, rv1

        init = jnp.full((_R, 1), _BIG, jnp.float32)
        rv0, rv1 = lax.fori_loop(0, nc, col_body, (init, init))
        part = jnp.float32(0.0)
        for (a_blk, a2, rvalid, ca), rv in zip(blks, (rv0, rv1)):
            mrow = jnp.where(rvalid, 1.0, 0.0)
            part = part + jnp.sum(jnp.maximum(rv + a2, 0.0) * mrow)
        return sum_ab + part

    sum_ab = lax.fori_loop(0, nr, row_body, jnp.float32(0.0))
    colvalid = jnp.where(iota < cnt_i, 1.0, 0.0)
    sum_ba = jnp.sum(jnp.maximum(colmin_s[:, :] + b2, 0.0) * colvalid)
    cd_b = (sum_ab + sum_ba) / cntf

    l1_b = l1s_ref[b, 0]
    for k in range(1, _L):
        l1_b = l1_b + l1s_ref[b, k]
    acc_ref[0] = acc_ref[0] + l1_b
    acc_ref[1] = acc_ref[1] + cntf
    acc_ref[2] = acc_ref[2] + cd_b

    @pl.when(b == n_batch - 1)
    def _emit():
        l1 = acc_ref[0] / (3.0 * acc_ref[1])
        cd = acc_ref[2] * (1.0 / n_batch)
        out_ref[0, 0] = 0.5 * (l1 + cd)


def _tc_chamfer(aCT, bCT, cnts, l1s, n_batch):
    return pl.pallas_call(
        functools.partial(_tc_chamfer_kernel, n_batch=n_batch),
        grid=(n_batch,),
        in_specs=[
            pl.BlockSpec((1, 8, _NP), lambda b: (b, 0, 0)),
            pl.BlockSpec((1, 8, _NP), lambda b: (b, 0, 0)),
            pl.BlockSpec(memory_space=pltpu.SMEM),
            pl.BlockSpec(memory_space=pltpu.SMEM),
        ],
        out_specs=pl.BlockSpec((1, 1), lambda b: (0, 0),
                               memory_space=pltpu.SMEM),
        out_shape=jax.ShapeDtypeStruct((1, 1), jnp.float32),
        scratch_shapes=[
            pltpu.VMEM((8, _N), jnp.float32),
            pltpu.VMEM((1, _N), jnp.float32),
            pltpu.VMEM((1, _N), jnp.float32),
            pltpu.VMEM((_N, 8), jnp.float32),
            pltpu.SMEM((4,), jnp.float32),
        ],
    )(aCT, bCT, cnts, l1s)


@jax.jit
def kernel(pred, target, mask, points):
    B, N, D = pred.shape
    predT = jnp.transpose(pred, (0, 2, 1)).reshape(B * D, N)
    targetT = jnp.transpose(target, (0, 2, 1)).reshape(B * D, N)
    pointsT = jnp.transpose(points, (0, 2, 1)).reshape(B * D, N)
    aCT_f, bCT_f, cnts, l1s = _sc_compact(predT, targetT, pointsT, mask)
    aCT = aCT_f.reshape(B, 8, _NP)
    bCT = bCT_f.reshape(B, 8, _NP)
    out = _tc_chamfer(aCT, bCT, cnts, l1s, B)
    return out[0, 0]


# speedup vs baseline: 1.3992x; 1.0359x over previous
import functools

import jax
import jax.numpy as jnp
from jax import lax
from jax.experimental import pallas as pl
from jax.experimental.pallas import tpu as pltpu
from jax.experimental.pallas import tpu_sc as plsc

_N = 4096
_B = 4
_R = 512
_C = 1024
_BIG = 1e10
_L = 16
_NP = _N + 128


def _sc_compact_body(predT_hbm, targetT_hbm, pointsT_hbm, mask_hbm,
                     aCT_hbm, bCT_hbm, cnt_hbm, l1_hbm,
                     src1, src2, maskv, posbuf, valbuf, zbuf,
                     stage_i, stage_f, shared, sem):
    wid = lax.axis_index("s") * 2 + lax.axis_index("c")
    batch = wid // 8
    role = wid % 8
    n_chunks = _N // _L

    liota = lax.iota(jnp.int32, _L)
    _dnums = lax.GatherDimensionNumbers(
        offset_dims=(), collapsed_slice_dims=(0,), start_index_map=(0,))

    def _vgather(x, idx):
        return lax.gather(x, idx[:, None], _dnums, slice_sizes=(1,),
                          mode=lax.GatherScatterMode.PROMISE_IN_BOUNDS)

    def _prefix_sum(x):
        for k in (1, 2, 4, 8):
            idx = jnp.maximum(liota - k, 0)
            g = _vgather(x, idx)
            x = x + jnp.where(liota >= k, g, 0)
        return x

    def _zero_fill():
        def zb(i, c):
            zbuf[pl.ds(i * _L, _L)] = jnp.zeros((_L,), jnp.float32)
            return c
        lax.fori_loop(0, _NP // _L, zb, 0, unroll=8)

    @pl.when(role < 7)
    def _load_mask():
        pltpu.sync_copy(mask_hbm.at[batch, :], maskv)

    def _compact(src2_hbm, dst_hbm, coord, emit_cnt):
        pltpu.sync_copy(pointsT_hbm.at[batch * 3 + coord, :], src1)
        pltpu.sync_copy(src2_hbm.at[batch * 3 + coord, :], src2)
        row0 = (batch * 8 + coord) * _NP
        sh0 = lax.axis_index("s") * _NP
        _zero_fill()
        pltpu.sync_copy(zbuf, shared.at[pl.ds(sh0, _NP)])

        def cbody(i, base):
            c0 = i * _L
            mv = maskv[pl.ds(c0, _L)]
            valid = mv > 0
            x = src1[pl.ds(c0, _L)] + src2[pl.ds(c0, _L)]
            pre = _prefix_sum(mv)
            pos = jnp.where(valid, base + pre - 1, _N + liota) + sh0
            posbuf[0, pl.ds(c0, _L)] = pos
            valbuf[0, pl.ds(c0, _L)] = x
            total = _vgather(pre, jnp.full((_L,), _L - 1, jnp.int32))
            return base + total
        cnt = lax.fori_loop(0, n_chunks, cbody, jnp.zeros((_L,), jnp.int32),
                            unroll=4)

        copies = []
        for j in range(_N // 128):
            pb = posbuf.at[0, pl.ds(j * 128, 128)]
            vb = valbuf.at[0, pl.ds(j * 128, 128)]
            copies.append(pltpu.async_copy(vb, shared.at[pb], sem))
        for c in copies:
            c.wait()
        pltpu.sync_copy(shared.at[pl.ds(sh0, _N)], src1)
        pltpu.sync_copy(src1, dst_hbm.at[pl.ds(row0, _N)])

        if emit_cnt:
            stage_i[pl.ds(0, _L)] = cnt
            pltpu.sync_copy(stage_i, cnt_hbm.at[batch, :])

    @pl.when(role < 3)
    def _a_side():
        _compact(targetT_hbm, aCT_hbm, role, False)

    @pl.when(role == 3)
    def _b_side0():
        _compact(predT_hbm, bCT_hbm, jnp.int32(0), True)

    @pl.when(jnp.logical_and(role >= 4, role < 6))
    def _b_side():
        _compact(predT_hbm, bCT_hbm, role - 3, False)

    @pl.when(role == 6)
    def _l1_and_zero_a():
        acc0 = jnp.zeros((_L,), jnp.float32)
        for k in range(3):
            pltpu.sync_copy(predT_hbm.at[batch * 3 + k, :], src1)
            pltpu.sync_copy(targetT_hbm.at[batch * 3 + k, :], src2)

            def l1body(i, acc):
                c0 = i * _L
                mv = maskv[pl.ds(c0, _L)].astype(jnp.float32)
                diff = jnp.abs(src1[pl.ds(c0, _L)] - src2[pl.ds(c0, _L)])
                return acc + diff * mv
            acc0 = lax.fori_loop(0, n_chunks, l1body, acc0, unroll=4)
        stage_f[pl.ds(0, _L)] = acc0
        pltpu.sync_copy(stage_f, l1_hbm.at[batch, :])

        _zero_fill()
        for k in range(3, 8):
            pltpu.sync_copy(zbuf, aCT_hbm.at[pl.ds((batch * 8 + k) * _NP,
                                                   _NP)])

    @pl.when(role == 7)
    def _zero_b():
        _zero_fill()
        for k in range(3, 8):
            pltpu.sync_copy(zbuf, bCT_hbm.at[pl.ds((batch * 8 + k) * _NP,
                                                   _NP)])


def _sc_compact(predT, targetT, pointsT, mask):
    return functools.partial(
        pl.kernel,
        mesh=plsc.VectorSubcoreMesh(core_axis_name="c", subcore_axis_name="s",
                                    num_cores=2, num_subcores=16),
        out_type=[
            jax.ShapeDtypeStruct((_B * 8 * _NP,), jnp.float32),
            jax.ShapeDtypeStruct((_B * 8 * _NP,), jnp.float32),
            jax.ShapeDtypeStruct((_B, _L), jnp.int32),
            jax.ShapeDtypeStruct((_B, _L), jnp.float32),
        ],
        scratch_types=[
            pltpu.VMEM((_N,), jnp.float32),
            pltpu.VMEM((_N,), jnp.float32),
            pltpu.VMEM((_N,), jnp.int32),
            pltpu.VMEM((1, _N), jnp.int32),
            pltpu.VMEM((1, _N), jnp.float32),
            pltpu.VMEM((_NP,), jnp.float32),
            pltpu.VMEM((_L,), jnp.int32),
            pltpu.VMEM((_L,), jnp.float32),
            pltpu.VMEM_SHARED((16 * _NP,), jnp.float32),
            pltpu.SemaphoreType.DMA,
        ],
    )(_sc_compact_body)(predT, targetT, pointsT, mask)


def _tc_chamfer_kernel(aCT_ref, bCT_ref, cnts_ref, l1s_ref, out_ref,
                       bneg_s, cb_s, colmin_s, aN_s, acc_ref, *, n_batch):
    b = pl.program_id(0)
    cnt_i = cnts_ref[b, 0]
    cntf = cnt_i.astype(jnp.float32)

    @pl.when(b == 0)
    def _init():
        acc_ref[0] = 0.0
        acc_ref[1] = 0.0
        acc_ref[2] = 0.0

    bpt = bCT_ref[0][:, :_N]
    b2 = jnp.sum(bpt * bpt, axis=0, keepdims=True)
    bneg_s[:, :] = -2.0 * bpt
    iota = lax.broadcasted_iota(jnp.int32, (1, _N), 1)
    cb_s[:, :] = b2 + jnp.where(iota < cnt_i, 0.0, _BIG)
    colmin_s[:, :] = jnp.full((1, _N), _BIG, jnp.float32)
    aN_s[:, :] = jnp.transpose(aCT_ref[0][:, :_N], (1, 0))

    nr = (cnt_i + 2 * _R - 1) // (2 * _R)
    nc = (cnt_i + 2 * _C - 1) // (2 * _C)

    def row_body(i, sum_ab):
        blks = []
        for v in range(2):
            r0 = i * (2 * _R) + v * _R
            a_blk = aN_s[pl.ds(r0, _R), :]
            a2 = jnp.sum(a_blk * a_blk, axis=1, keepdims=True)
            riota = lax.broadcasted_iota(jnp.int32, (_R, 1), 0) + r0
            rvalid = riota < cnt_i
            ca = a2 + jnp.where(rvalid, 0.0, _BIG)
            blks.append((a_blk, a2, rvalid, ca))

        def col_body(j, rvmins):
            rv0, rv1 = rvmins
            c0 = j * (2 * _C)
            for u in range(2):
                cu = c0 + u * _C
                bn = bneg_s[:, pl.ds(cu, _C)]
                cbj = cb_s[:, pl.ds(cu, _C)]
                cvs = []
                for v, (a_blk, a2, rvalid, ca) in enumerate(blks):
                    s = lax.dot_general(a_blk, bn, (((1,), (0,)), ((), ())),
                                        preferred_element_type=jnp.float32)
                    rv = jnp.min(s + cbj, axis=1, keepdims=True)
                    if v == 0:
                        rv0 = jnp.minimum(rv0, rv)
                    else:
                        rv1 = jnp.minimum(rv1, rv)
                    cvs.append(jnp.min(s + ca, axis=0, keepdims=True))
                cv = jnp.minimum(cvs[0], cvs[1])
                colmin_s[:, pl.ds(cu, _C)] = jnp.minimum(
                    colmin_s[:, pl.ds(cu, _C)], cv)
            return rv0, rv1

        init = jnp.full((_R, 1), _BIG, jnp.float32)
        rv0, rv1 = lax.fori_loop(0, nc, col_body, (init, init))
        part = jnp.float32(0.0)
        for (a_blk, a2, rvalid, ca), rv in zip(blks, (rv0, rv1)):
            mrow = jnp.where(rvalid, 1.0, 0.0)
            part = part + jnp.sum(jnp.maximum(rv + a2, 0.0) * mrow)
        return sum_ab + part

    sum_ab = lax.fori_loop(0, nr, row_body, jnp.float32(0.0))
    colvalid = jnp.where(iota < cnt_i, 1.0, 0.0)
    sum_ba = jnp.sum(jnp.maximum(colmin_s[:, :] + b2, 0.0) * colvalid)
    cd_b = (sum_ab + sum_ba) / cntf

    l1_b = l1s_ref[b, 0]
    for k in range(1, _L):
        l1_b = l1_b + l1s_ref[b, k]
    acc_ref[0] = acc_ref[0] + l1_b
    acc_ref[1] = acc_ref[1] + cntf
    acc_ref[2] = acc_ref[2] + cd_b

    @pl.when(b == n_batch - 1)
    def _emit():
        l1 = acc_ref[0] / (3.0 * acc_ref[1])
        cd = acc_ref[2] * (1.0 / n_batch)
        out_ref[0, 0] = 0.5 * (l1 + cd)


def _tc_chamfer(aCT, bCT, cnts, l1s, n_batch):
    return pl.pallas_call(
        functools.partial(_tc_chamfer_kernel, n_batch=n_batch),
        grid=(n_batch,),
        in_specs=[
            pl.BlockSpec((1, 8, _NP), lambda b: (b, 0, 0)),
            pl.BlockSpec((1, 8, _NP), lambda b: (b, 0, 0)),
            pl.BlockSpec(memory_space=pltpu.SMEM),
            pl.BlockSpec(memory_space=pltpu.SMEM),
        ],
        out_specs=pl.BlockSpec((1, 1), lambda b: (0, 0),
                               memory_space=pltpu.SMEM),
        out_shape=jax.ShapeDtypeStruct((1, 1), jnp.float32),
        scratch_shapes=[
            pltpu.VMEM((8, _N), jnp.float32),
            pltpu.VMEM((1, _N), jnp.float32),
            pltpu.VMEM((1, _N), jnp.float32),
            pltpu.VMEM((_N, 8), jnp.float32),
            pltpu.SMEM((4,), jnp.float32),
        ],
    )(aCT, bCT, cnts, l1s)


@jax.jit
def kernel(pred, target, mask, points):
    B, N, D = pred.shape
    predT = jnp.transpose(pred, (0, 2, 1)).reshape(B * D, N)
    targetT = jnp.transpose(target, (0, 2, 1)).reshape(B * D, N)
    pointsT = jnp.transpose(points, (0, 2, 1)).reshape(B * D, N)
    aCT_f, bCT_f, cnts, l1s = _sc_compact(predT, targetT, pointsT, mask)
    aCT = aCT_f.reshape(B, 8, _NP)
    bCT = bCT_f.reshape(B, 8, _NP)
    out = _tc_chamfer(aCT, bCT, cnts, l1s, B)
    return out[0, 0]
